# Initial kernel scaffold; baseline (speedup 1.0000x reference)
import jax
import jax.numpy as jnp
from jax.experimental import pallas as pl


def kernel(x, edge_index, W1, b1, W2, b2, W_ih, W_hh, b_ih, b_hh, W_fc, b_fc):
    raise NotImplementedError("write your pallas kernel here")


# speedup vs baseline: 6.0984x; 6.0984x over previous
import functools

import jax
import jax.numpy as jnp
from jax import lax
from jax.experimental import pallas as pl
from jax.experimental.pallas import tpu as pltpu
from jax.experimental.pallas import tpu_sc as plsc

N = 10000
E = 320000
F = 128
EMB = 256
VOCAB = 10000

NP = 10240
CH = 128
NCH = 80
EP = 32 * NCH * CH
NTILE = 16
RT = NP // NTILE
NBUF = 2


def _sc_mesh():
    return plsc.VectorSubcoreMesh(core_axis_name="c", subcore_axis_name="s")


def _deg_partials(dstp, zinit, ones):
    @functools.partial(
        pl.kernel,
        out_type=jax.ShapeDtypeStruct((2, NP, F), jnp.float32),
        mesh=_sc_mesh(),
        scratch_types=[
            pltpu.VMEM((NCH, CH), jnp.int32),
            pltpu.VMEM((CH, F), jnp.float32),
            pltpu.VMEM_SHARED((NP, F), jnp.float32),
        ],
    )
    def k(dst_hbm, z_hbm, ones_hbm, out_hbm, dall, ones_v, acc_sh):
        cid = lax.axis_index("c")
        sid = lax.axis_index("s")
        wid = cid * 16 + sid
        r0 = sid * RT
        pltpu.sync_copy(z_hbm.at[pl.ds(r0, RT)], acc_sh.at[pl.ds(r0, RT)])
        pltpu.sync_copy(ones_hbm, ones_v)
        pltpu.sync_copy(dst_hbm.at[wid], dall)
        plsc.subcore_barrier()

        def body(i, carry):
            pltpu.sync_copy(ones_v, acc_sh.at[dall.at[i]], add=True)
            return carry

        lax.fori_loop(0, NCH, body, 0)
        plsc.subcore_barrier()
        pltpu.sync_copy(acc_sh.at[pl.ds(r0, RT)], out_hbm.at[cid, pl.ds(r0, RT)])

    return k(dstp, zinit, ones)


def _aggregate(y, srcp, dstp, init):
    @functools.partial(
        pl.kernel,
        out_type=jax.ShapeDtypeStruct((2, NP, F), jnp.float32),
        mesh=_sc_mesh(),
        scratch_types=(
            [pltpu.VMEM((NCH, CH), jnp.int32)]
            + [pltpu.VMEM((CH,), jnp.int32) for _ in range(NBUF)]
            + [pltpu.VMEM((CH, F), jnp.float32) for _ in range(NBUF)]
            + [pltpu.VMEM_SHARED((NP, F), jnp.float32)]
            + [pltpu.SemaphoreType.DMA for _ in range(NBUF)]
        ),
    )
    def k(y_hbm, src_hbm, dst_hbm, init_hbm, out_hbm, *scr):
        dall = scr[0]
        sidx = scr[1:1 + NBUF]
        rows = scr[1 + NBUF:1 + 2 * NBUF]
        acc_sh = scr[1 + 2 * NBUF]
        sems = scr[2 + 2 * NBUF:]
        cid = lax.axis_index("c")
        sid = lax.axis_index("s")
        wid = cid * 16 + sid
        r0 = sid * RT
        base = wid * (NCH * CH)
        pltpu.sync_copy(init_hbm.at[cid, pl.ds(r0, RT)], acc_sh.at[pl.ds(r0, RT)])
        pltpu.sync_copy(dst_hbm.at[wid], dall)
        plsc.subcore_barrier()

        for b in range(NBUF):
            pltpu.sync_copy(src_hbm.at[pl.ds(base + b * CH, CH)], sidx[b])
            pltpu.async_copy(y_hbm.at[sidx[b]], rows[b], sems[b])

        def body(g, carry):
            for b in range(NBUF):
                i = g * NBUF + b
                pltpu.make_async_copy(y_hbm.at[sidx[b]], rows[b],
                                      sems[b]).wait()
                pltpu.sync_copy(rows[b], acc_sh.at[dall.at[i]], add=True)

                @pl.when(i + NBUF < NCH)
                def _():
                    pltpu.sync_copy(
                        src_hbm.at[pl.ds(base + (i + NBUF) * CH, CH)], sidx[b])
                    pltpu.async_copy(y_hbm.at[sidx[b]], rows[b], sems[b])
            return carry

        lax.fori_loop(0, NCH // NBUF, body, 0)
        plsc.subcore_barrier()
        pltpu.sync_copy(acc_sh.at[pl.ds(r0, RT)], out_hbm.at[cid, pl.ds(r0, RT)])

    return k(y, srcp, dstp, init)


def _cdiv(a, b):
    return (a + b - 1) // b


def _mm_scale(x, W1, deg_p):
    bm = 512
    grid = (_cdiv(NP, bm),)

    def body(x_ref, w_ref, d_ref, y_ref):
        dinv = lax.rsqrt(d_ref[0, :, 0:1] + d_ref[1, :, 0:1] + 1.0)
        xw = jnp.dot(x_ref[...], w_ref[...], preferred_element_type=jnp.float32)
        y_ref[...] = xw * dinv

    return pl.pallas_call(
        body,
        grid=grid,
        in_specs=[
            pl.BlockSpec((bm, F), lambda i: (i, 0)),
            pl.BlockSpec((F, F), lambda i: (0, 0)),
            pl.BlockSpec((2, bm, F), lambda i: (0, i, 0)),
        ],
        out_specs=pl.BlockSpec((bm, F), lambda i: (i, 0)),
        out_shape=jax.ShapeDtypeStruct((NP, F), jnp.float32),
    )(x, W1, deg_p)


def _post_mm(acc1, deg_p, b1r, W2):
    bm = 512
    grid = (_cdiv(NP, bm),)

    def body(a_ref, d_ref, b_ref, w_ref, y_ref):
        dinv = lax.rsqrt(d_ref[0, :, 0:1] + d_ref[1, :, 0:1] + 1.0)
        accf = a_ref[0] + a_ref[1]
        h = jnp.maximum(accf * dinv + b_ref[...], 0.0)
        y_ref[...] = jnp.dot(h, w_ref[...], preferred_element_type=jnp.float32) * dinv

    return pl.pallas_call(
        body,
        grid=grid,
        in_specs=[
            pl.BlockSpec((2, bm, F), lambda i: (0, i, 0)),
            pl.BlockSpec((2, bm, F), lambda i: (0, i, 0)),
            pl.BlockSpec((1, F), lambda i: (0, 0)),
            pl.BlockSpec((F, F), lambda i: (0, 0)),
        ],
        out_specs=pl.BlockSpec((bm, F), lambda i: (i, 0)),
        out_shape=jax.ShapeDtypeStruct((NP, F), jnp.float32),
    )(acc1, deg_p, b1r, W2)


def _final(acc2, deg_p, b2r, W_ih, bgr, W_fc, bfcr):
    bm = 256
    bn = 1024
    grid = (_cdiv(N, bm), _cdiv(VOCAB, bn))

    def body(a_ref, d_ref, b2_ref, wih_ref, bg_ref, wfc_ref, bfc_ref,
             out_ref, hh_ref):
        j = pl.program_id(1)

        @pl.when(j == 0)
        def _():
            dinv = lax.rsqrt(d_ref[0, :, 0:1] + d_ref[1, :, 0:1] + 1.0)
            h2 = (a_ref[0] + a_ref[1]) * dinv + b2_ref[...]
            gates = lax.dot_general(
                h2, wih_ref[...], (((1,), (1,)), ((), ())),
                preferred_element_type=jnp.float32) + bg_ref[...]
            ii = jax.nn.sigmoid(gates[:, 0:EMB])
            gg = jnp.tanh(gates[:, 2 * EMB:3 * EMB])
            oo = jax.nn.sigmoid(gates[:, 3 * EMB:4 * EMB])
            hh_ref[...] = oo * jnp.tanh(ii * gg)

        out_ref[...] = lax.dot_general(
            hh_ref[...], wfc_ref[...], (((1,), (1,)), ((), ())),
            preferred_element_type=jnp.float32) + bfc_ref[...]

    return pl.pallas_call(
        body,
        grid=grid,
        in_specs=[
            pl.BlockSpec((2, bm, F), lambda i, j: (0, i, 0)),
            pl.BlockSpec((2, bm, F), lambda i, j: (0, i, 0)),
            pl.BlockSpec((1, F), lambda i, j: (0, 0)),
            pl.BlockSpec((4 * EMB, F), lambda i, j: (0, 0)),
            pl.BlockSpec((1, 4 * EMB), lambda i, j: (0, 0)),
            pl.BlockSpec((bn, EMB), lambda i, j: (j, 0)),
            pl.BlockSpec((1, bn), lambda i, j: (0, j)),
        ],
        out_specs=pl.BlockSpec((bm, bn), lambda i, j: (i, j)),
        out_shape=jax.ShapeDtypeStruct((N, VOCAB), jnp.float32),
        scratch_shapes=[pltpu.VMEM((bm, EMB), jnp.float32)],
    )(acc2, deg_p, b2r, W_ih, bgr, W_fc, bfcr)


def kernel(x, edge_index, W1, b1, W2, b2, W_ih, W_hh, b_ih, b_hh, W_fc, b_fc):
    src = edge_index[0].astype(jnp.int32)
    dst = edge_index[1].astype(jnp.int32)
    pad = EP - E
    srcp = jnp.concatenate([src, jnp.full((pad,), N, jnp.int32)])
    dstp = jnp.concatenate(
        [dst, N + (jnp.arange(pad, dtype=jnp.int32) % 16)])
    dstp = dstp.reshape(32, NCH, CH)

    zinit = jnp.zeros((NP, F), jnp.float32)
    ones = jnp.ones((CH, F), jnp.float32)
    deg_p = _deg_partials(dstp, zinit, ones)

    zacc = jnp.zeros((1, NP, F), jnp.float32)
    y1 = _mm_scale(x, W1, deg_p)
    acc1 = _aggregate(y1, srcp, dstp,
                      jnp.concatenate([y1[None], zacc], axis=0))

    y2 = _post_mm(acc1, deg_p, b1.reshape(1, F), W2)
    acc2 = _aggregate(y2, srcp, dstp,
                      jnp.concatenate([y2[None], zacc], axis=0))

    logits = _final(acc2, deg_p, b2.reshape(1, F), W_ih,
                    (b_ih + b_hh).reshape(1, 4 * EMB), W_fc,
                    b_fc.reshape(1, VOCAB))
    return logits
